# baseline (speedup 1.0000x reference)
import functools

import jax
import jax.numpy as jnp
from jax import lax
from jax.experimental import pallas as pl
from jax.experimental.pallas import tpu as pltpu
from jax.experimental.pallas import tpu_sc as plsc

_NC = 2
_NS = 16
_NW = _NC * _NS
_CH = 128
_CHG = 64
_DEPTH = 3


def _mesh():
    return plsc.VectorSubcoreMesh(
        core_axis_name="c", subcore_axis_name="s",
        num_cores=_NC, num_subcores=_NS)


def _part(n_nodes):
    base = n_nodes // (8 * _NS) * 8
    tail = n_nodes - base * _NS
    return base, tail


def _init_acc(minit_c, acc_sh, sid, n_nodes):
    rpt, tail = _part(n_nodes)
    pltpu.sync_copy(minit_c.at[pl.ds(sid * rpt, rpt)],
                    acc_sh.at[pl.ds(sid * rpt, rpt)])
    if tail:
        @pl.when(sid == _NS - 1)
        def _():
            pltpu.sync_copy(minit_c.at[pl.ds(rpt * _NS, tail)],
                            acc_sh.at[pl.ds(rpt * _NS, tail)])


def _write_out(acc_sh, out_ref, sid, n_nodes):
    rpt, tail = _part(n_nodes)
    pltpu.sync_copy(acc_sh.at[pl.ds(sid * rpt, rpt)],
                    out_ref.at[pl.ds(sid * rpt, rpt)])
    if tail:
        @pl.when(sid == _NS - 1)
        def _():
            pltpu.sync_copy(acc_sh.at[pl.ds(rpt * _NS, tail)],
                            out_ref.at[pl.ds(rpt * _NS, tail)])


def _make_seg_sum(n_nodes, d, n_edges):
    ncht = n_edges // _CH
    nb = ncht // _NW
    nrem = ncht - nb * _NW
    npair = nb // 2
    assert nb % 2 == 0 and n_edges == ncht * _CH

    @functools.partial(
        pl.kernel,
        out_type=jax.ShapeDtypeStruct((_NC, n_nodes, d), jnp.float32),
        mesh=_mesh(),
        scratch_types=[
            pltpu.VMEM((2, _CH), jnp.int32),
            pltpu.VMEM((2, _CH), jnp.int32),
            pltpu.VMEM((_CH, d), jnp.float32),
            pltpu.VMEM((_CH, d), jnp.float32),
            pltpu.VMEM_SHARED((n_nodes, d), jnp.float32),
            pltpu.SemaphoreType.DMA,
            pltpu.SemaphoreType.DMA,
        ],
    )
    def seg_sum(t_hbm, sd_hbm, minit_hbm, out_hbm,
                ib0, ib1, rows0, rows1, acc_sh, gsem0, gsem1):
        cid = lax.axis_index("c")
        sid = lax.axis_index("s")
        wid = cid * _NS + sid
        base = wid * nb

        _init_acc(minit_hbm.at[cid], acc_sh, sid, n_nodes)
        plsc.subcore_barrier()

        if nrem:
            @pl.when(wid < nrem)
            def _():
                pltpu.sync_copy(sd_hbm.at[nb * _NW + wid], ib0)
                pltpu.async_copy(t_hbm.at[ib0.at[0]], rows0, gsem0).wait()
                pltpu.sync_copy(rows0, acc_sh.at[ib0.at[1]], add=True)

        pltpu.sync_copy(sd_hbm.at[base], ib0)
        pltpu.sync_copy(sd_hbm.at[base + 1], ib1)
        pltpu.async_copy(t_hbm.at[ib0.at[0]], rows0, gsem0)

        def pair(g, _):
            i0 = base + 2 * g
            pltpu.make_async_copy(t_hbm.at[ib0.at[0]], rows0, gsem0).wait()
            pltpu.async_copy(t_hbm.at[ib1.at[0]], rows1, gsem1)
            pltpu.sync_copy(rows0, acc_sh.at[ib0.at[1]], add=True)

            @pl.when(g < npair - 1)
            def _():
                pltpu.sync_copy(sd_hbm.at[i0 + 2], ib0)

            pltpu.make_async_copy(t_hbm.at[ib1.at[0]], rows1, gsem1).wait()

            @pl.when(g < npair - 1)
            def _():
                pltpu.async_copy(t_hbm.at[ib0.at[0]], rows0, gsem0)

            pltpu.sync_copy(rows1, acc_sh.at[ib1.at[1]], add=True)

            @pl.when(g < npair - 1)
            def _():
                pltpu.sync_copy(sd_hbm.at[i0 + 3], ib1)
            return 0

        lax.fori_loop(0, npair, pair, 0)

        plsc.subcore_barrier()
        _write_out(acc_sh, out_hbm.at[cid], sid, n_nodes)

    return seg_sum


def _make_seg_gl(n_nodes, d, n_edges):
    ch = _CHG
    ncht = n_edges // ch
    nb = ncht // _NW
    nrem = ncht - nb * _NW
    npair = nb // 2
    assert nb % 2 == 0 and n_edges == ncht * ch

    @functools.partial(
        pl.kernel,
        out_type=jax.ShapeDtypeStruct((_NC, n_nodes, d), jnp.float32),
        mesh=_mesh(),
        scratch_types=[
            pltpu.VMEM((2, ch), jnp.int32),
            pltpu.VMEM((2, ch), jnp.int32),
            pltpu.VMEM((ch, d), jnp.float32),
            pltpu.VMEM((ch, d), jnp.float32),
            pltpu.VMEM((ch, d), jnp.float32),
            pltpu.VMEM((ch, d), jnp.float32),
            pltpu.VMEM_SHARED((n_nodes, d), jnp.float32),
            pltpu.SemaphoreType.DMA,
            pltpu.SemaphoreType.DMA,
            pltpu.SemaphoreType.DMA,
            pltpu.SemaphoreType.DMA,
        ],
    )
    def seg_gl(g_hbm, a_hbm, sd_hbm, minit_hbm, out_hbm,
               ib0, ib1, rg0, rg1, ra0, ra1, acc_sh,
               sg0, sg1, sa0, sa1):
        cid = lax.axis_index("c")
        sid = lax.axis_index("s")
        wid = cid * _NS + sid
        base = wid * nb

        _init_acc(minit_hbm.at[cid], acc_sh, sid, n_nodes)
        plsc.subcore_barrier()

        def _arows(q):
            return a_hbm.at[pl.ds(q * ch, ch)]

        if nrem:
            @pl.when(wid < nrem)
            def _():
                q = nb * _NW + wid
                pltpu.sync_copy(sd_hbm.at[q], ib0)
                pltpu.async_copy(g_hbm.at[ib0.at[0]], rg0, sg0).wait()
                pltpu.sync_copy(_arows(q), ra0)
                pltpu.sync_copy(rg0, acc_sh.at[ib0.at[1]], add=True)
                pltpu.sync_copy(ra0, acc_sh.at[ib0.at[1]], add=True)

        pltpu.sync_copy(sd_hbm.at[base], ib0)
        pltpu.sync_copy(sd_hbm.at[base + 1], ib1)
        pltpu.async_copy(g_hbm.at[ib0.at[0]], rg0, sg0)
        pltpu.async_copy(_arows(base), ra0, sa0)

        def pair(g, _):
            i0 = base + 2 * g
            pltpu.make_async_copy(g_hbm.at[ib0.at[0]], rg0, sg0).wait()
            pltpu.make_async_copy(_arows(i0), ra0, sa0).wait()
            pltpu.async_copy(g_hbm.at[ib1.at[0]], rg1, sg1)
            pltpu.async_copy(_arows(i0 + 1), ra1, sa1)
            pltpu.sync_copy(rg0, acc_sh.at[ib0.at[1]], add=True)
            pltpu.sync_copy(ra0, acc_sh.at[ib0.at[1]], add=True)

            @pl.when(g < npair - 1)
            def _():
                pltpu.sync_copy(sd_hbm.at[i0 + 2], ib0)

            pltpu.make_async_copy(g_hbm.at[ib1.at[0]], rg1, sg1).wait()
            pltpu.make_async_copy(_arows(i0 + 1), ra1, sa1).wait()

            @pl.when(g < npair - 1)
            def _():
                pltpu.async_copy(g_hbm.at[ib0.at[0]], rg0, sg0)
                pltpu.async_copy(_arows(i0 + 2), ra0, sa0)

            pltpu.sync_copy(rg1, acc_sh.at[ib1.at[1]], add=True)
            pltpu.sync_copy(ra1, acc_sh.at[ib1.at[1]], add=True)

            @pl.when(g < npair - 1)
            def _():
                pltpu.sync_copy(sd_hbm.at[i0 + 3], ib1)
            return 0

        lax.fori_loop(0, npair, pair, 0)

        plsc.subcore_barrier()
        _write_out(acc_sh, out_hbm.at[cid], sid, n_nodes)

    return seg_gl


_BR = 1000
_BE = 2000


def _edge_body(e_ref, w_ref, b_ref, out_ref):
    a = jnp.dot(e_ref[...], w_ref[...], preferred_element_type=jnp.float32)
    out_ref[...] = a + b_ref[...]


def _tc_edge_msg(E, We, b):
    ne, de = E.shape
    dh = We.shape[1]
    grid = ne // _BE
    return pl.pallas_call(
        _edge_body,
        grid=(grid,),
        in_specs=[
            pl.BlockSpec((_BE, de), lambda i: (i, 0)),
            pl.BlockSpec((de, dh), lambda i: (0, 0)),
            pl.BlockSpec((1, dh), lambda i: (0, 0)),
        ],
        out_specs=pl.BlockSpec((_BE, dh), lambda i: (i, 0)),
        out_shape=jax.ShapeDtypeStruct((ne, dh), jnp.float32),
    )(E, We, b.reshape(1, dh))


def _init2_body(v_ref, wi_ref, b_ref, wp_ref, h0_ref, g0_ref):
    h0 = jnp.dot(v_ref[...], wi_ref[...],
                 preferred_element_type=jnp.float32)
    h0 = jnp.maximum(h0 + b_ref[...], 0.0)
    h0_ref[...] = h0
    g0_ref[...] = jnp.dot(h0, wp_ref[...],
                          preferred_element_type=jnp.float32)


def _tc_init2(V, Wi, b, Wp):
    n, dv = V.shape
    dh = Wi.shape[1]
    grid = n // _BR
    return pl.pallas_call(
        _init2_body,
        grid=(grid,),
        in_specs=[
            pl.BlockSpec((_BR, dv), lambda i: (i, 0)),
            pl.BlockSpec((dv, dh), lambda i: (0, 0)),
            pl.BlockSpec((1, dh), lambda i: (0, 0)),
            pl.BlockSpec((dh, dh), lambda i: (0, 0)),
        ],
        out_specs=[
            pl.BlockSpec((_BR, dh), lambda i: (i, 0)),
            pl.BlockSpec((_BR, dh), lambda i: (i, 0)),
        ],
        out_shape=[
            jax.ShapeDtypeStruct((n, dh), jnp.float32),
            jax.ShapeDtypeStruct((n, dh), jnp.float32),
        ],
    )(V, Wi, b.reshape(1, dh), Wp)


def _hd_body(h0_ref, hp_ref, m_ref, wp_ref, h_ref, d_ref):
    h = jnp.maximum(h0_ref[...] + m_ref[0] + m_ref[1], 0.0)
    h_ref[...] = h
    d_ref[...] = jnp.dot(h - hp_ref[...], wp_ref[...],
                         preferred_element_type=jnp.float32)


def _tc_hd(H0, Hprev, M, Wp):
    n, dh = H0.shape
    grid = n // _BR
    return pl.pallas_call(
        _hd_body,
        grid=(grid,),
        in_specs=[
            pl.BlockSpec((_BR, dh), lambda i: (i, 0)),
            pl.BlockSpec((_BR, dh), lambda i: (i, 0)),
            pl.BlockSpec((_NC, _BR, dh), lambda i: (0, i, 0)),
            pl.BlockSpec((dh, dh), lambda i: (0, 0)),
        ],
        out_specs=[
            pl.BlockSpec((_BR, dh), lambda i: (i, 0)),
            pl.BlockSpec((_BR, dh), lambda i: (i, 0)),
        ],
        out_shape=[
            jax.ShapeDtypeStruct((n, dh), jnp.float32),
            jax.ShapeDtypeStruct((n, dh), jnp.float32),
        ],
    )(H0, Hprev, M, Wp)


def _h2_body(h0_ref, m_ref, out_ref):
    out_ref[...] = jnp.maximum(h0_ref[...] + m_ref[0] + m_ref[1], 0.0)


def _tc_h2(H0, M):
    n, dh = H0.shape
    grid = n // _BR
    return pl.pallas_call(
        _h2_body,
        grid=(grid,),
        in_specs=[
            pl.BlockSpec((_BR, dh), lambda i: (i, 0)),
            pl.BlockSpec((_NC, _BR, dh), lambda i: (0, i, 0)),
        ],
        out_specs=pl.BlockSpec((_BR, dh), lambda i: (i, 0)),
        out_shape=jax.ShapeDtypeStruct((n, dh), jnp.float32),
    )(H0, M)


def _final_body(v_ref, wv_ref, p_ref, wp_ref, b_ref, out_ref):
    p = p_ref[0] + p_ref[1]
    h = jnp.dot(v_ref[...], wv_ref[...], preferred_element_type=jnp.float32)
    h = h + jnp.dot(p, wp_ref[...], preferred_element_type=jnp.float32)
    out_ref[...] = jnp.maximum(h + b_ref[...], 0.0)


def _tc_final(V, Wv, P, Wp, b):
    n, dv = V.shape
    dh = Wv.shape[1]
    grid = n // _BR
    return pl.pallas_call(
        _final_body,
        grid=(grid,),
        in_specs=[
            pl.BlockSpec((_BR, dv), lambda i: (i, 0)),
            pl.BlockSpec((dv, dh), lambda i: (0, 0)),
            pl.BlockSpec((_NC, _BR, dh), lambda i: (0, i, 0)),
            pl.BlockSpec((dh, dh), lambda i: (0, 0)),
            pl.BlockSpec((1, dh), lambda i: (0, 0)),
        ],
        out_specs=pl.BlockSpec((_BR, dh), lambda i: (i, 0)),
        out_shape=jax.ShapeDtypeStruct((n, dh), jnp.float32),
    )(V, Wv, P, Wp, b.reshape(1, dh))


def kernel(V, E, edge_index, rev_edge_index, batch, W_i, b_i, W_h, b_h,
           W_o, b_o):
    n, dv = V.shape
    ne, de = E.shape
    dh = W_i.shape[1]
    src = edge_index[0]
    dst = edge_index[1]
    Wp = W_h[:dh]

    sd128 = jnp.stack([src.reshape(ne // _CH, _CH),
                       dst.reshape(ne // _CH, _CH)], axis=1)
    sd64 = jnp.stack([src.reshape(ne // _CHG, _CHG),
                      dst.reshape(ne // _CHG, _CHG)], axis=1)
    zeros2 = jnp.zeros((_NC, n, dh), jnp.float32)

    seg_gl = _make_seg_gl(n, dh, ne)
    seg_sum = _make_seg_sum(n, dh, ne)

    A = _tc_edge_msg(E, W_h[dh:], b_h)
    H0, G0 = _tc_init2(V, W_i, b_i, Wp)

    M = seg_gl(G0, A, sd64, zeros2)

    Hprev = H0
    for _ in range(_DEPTH - 2):
        Hk, D = _tc_hd(H0, Hprev, M, Wp)
        M = seg_sum(D, sd128, M)
        Hprev = Hk

    Hlast = _tc_h2(H0, M)
    P = seg_sum(Hlast, sd128, zeros2)
    return _tc_final(V, W_o[:dv], P, W_o[dv:], b_o)


# speedup vs baseline: 1.2331x; 1.2331x over previous
import functools

import jax
import jax.numpy as jnp
from jax import lax
from jax.experimental import pallas as pl
from jax.experimental.pallas import tpu as pltpu
from jax.experimental.pallas import tpu_sc as plsc

_NC = 2
_NS = 16
_NW = _NC * _NS
_CH = 128
_DEPTH = 3


def _mesh():
    return plsc.VectorSubcoreMesh(
        core_axis_name="c", subcore_axis_name="s",
        num_cores=_NC, num_subcores=_NS)


def _part(n_nodes):
    base = n_nodes // (8 * _NS) * 8
    tail = n_nodes - base * _NS
    return base, tail


def _zero_acc(zeros_v, acc_sh, sid, n_nodes):
    rpt, tail = _part(n_nodes)
    start = sid * rpt
    for off in range(0, rpt, _CH):
        sz = min(_CH, rpt - off)
        pltpu.sync_copy(zeros_v.at[pl.ds(0, sz)],
                        acc_sh.at[pl.ds(start + off, sz)])
    if tail:
        @pl.when(sid == _NS - 1)
        def _():
            pltpu.sync_copy(zeros_v.at[pl.ds(0, tail)],
                            acc_sh.at[pl.ds(rpt * _NS, tail)])


def _write_out(acc_sh, out_ref, sid, n_nodes):
    rpt, tail = _part(n_nodes)
    pltpu.sync_copy(acc_sh.at[pl.ds(sid * rpt, rpt)],
                    out_ref.at[pl.ds(sid * rpt, rpt)])
    if tail:
        @pl.when(sid == _NS - 1)
        def _():
            pltpu.sync_copy(acc_sh.at[pl.ds(rpt * _NS, tail)],
                            out_ref.at[pl.ds(rpt * _NS, tail)])


def _make_seg_sum(n_nodes, d, n_edges):
    ncht = n_edges // _CH
    nb = ncht // _NW
    nrem = ncht - nb * _NW
    npair = nb // 2
    assert nb % 2 == 0 and n_edges == ncht * _CH

    @functools.partial(
        pl.kernel,
        out_type=jax.ShapeDtypeStruct((_NC, n_nodes, d), jnp.float32),
        mesh=_mesh(),
        scratch_types=[
            pltpu.VMEM((2, _CH), jnp.int32),
            pltpu.VMEM((2, _CH), jnp.int32),
            pltpu.VMEM((_CH, d), jnp.float32),
            pltpu.VMEM((_CH, d), jnp.float32),
            pltpu.VMEM_SHARED((n_nodes, d), jnp.float32),
            pltpu.SemaphoreType.DMA,
            pltpu.SemaphoreType.DMA,
        ],
    )
    def seg_sum(h_hbm, sd_hbm, zeros_hbm, out_hbm,
                ib0, ib1, rows0, rows1, acc_sh, gsem0, gsem1):
        cid = lax.axis_index("c")
        sid = lax.axis_index("s")
        wid = cid * _NS + sid
        base = wid * nb

        pltpu.sync_copy(zeros_hbm, rows0)
        _zero_acc(rows0, acc_sh, sid, n_nodes)
        plsc.subcore_barrier()

        if nrem:
            @pl.when(wid < nrem)
            def _():
                pltpu.sync_copy(sd_hbm.at[nb * _NW + wid], ib0)
                pltpu.async_copy(h_hbm.at[ib0.at[0]], rows0, gsem0).wait()
                pltpu.sync_copy(rows0, acc_sh.at[ib0.at[1]], add=True)

        pltpu.sync_copy(sd_hbm.at[base], ib0)
        pltpu.sync_copy(sd_hbm.at[base + 1], ib1)
        pltpu.async_copy(h_hbm.at[ib0.at[0]], rows0, gsem0)

        def pair(g, _):
            i0 = base + 2 * g
            pltpu.make_async_copy(h_hbm.at[ib0.at[0]], rows0, gsem0).wait()
            pltpu.async_copy(h_hbm.at[ib1.at[0]], rows1, gsem1)
            pltpu.sync_copy(rows0, acc_sh.at[ib0.at[1]], add=True)

            @pl.when(g < npair - 1)
            def _():
                pltpu.sync_copy(sd_hbm.at[i0 + 2], ib0)

            pltpu.make_async_copy(h_hbm.at[ib1.at[0]], rows1, gsem1).wait()

            @pl.when(g < npair - 1)
            def _():
                pltpu.async_copy(h_hbm.at[ib0.at[0]], rows0, gsem0)

            pltpu.sync_copy(rows1, acc_sh.at[ib1.at[1]], add=True)

            @pl.when(g < npair - 1)
            def _():
                pltpu.sync_copy(sd_hbm.at[i0 + 3], ib1)
            return 0

        lax.fori_loop(0, npair, pair, 0)

        plsc.subcore_barrier()
        _write_out(acc_sh, out_hbm.at[cid], sid, n_nodes)

    return seg_sum


def _make_lin_seg(n_nodes, d, n_edges):
    ncht = n_edges // _CH
    nb = ncht // _NW
    nrem = ncht - nb * _NW
    npair = nb // 2
    assert nb % 2 == 0 and n_edges == ncht * _CH

    @functools.partial(
        pl.kernel,
        out_type=jax.ShapeDtypeStruct((_NC, n_nodes, d), jnp.float32),
        mesh=_mesh(),
        scratch_types=[
            pltpu.VMEM((2, _CH), jnp.int32),
            pltpu.VMEM((2, _CH), jnp.int32),
            pltpu.VMEM((_CH, d), jnp.float32),
            pltpu.VMEM((_CH, d), jnp.float32),
            pltpu.VMEM_SHARED((n_nodes, d), jnp.float32),
            pltpu.SemaphoreType.DMA,
            pltpu.SemaphoreType.DMA,
        ],
    )
    def lin_seg(a_hbm, sd_hbm, zeros_hbm, out_hbm,
                ib0, ib1, rows0, rows1, acc_sh, gsem0, gsem1):
        cid = lax.axis_index("c")
        sid = lax.axis_index("s")
        wid = cid * _NS + sid
        base = wid * nb

        pltpu.sync_copy(zeros_hbm, rows0)
        _zero_acc(rows0, acc_sh, sid, n_nodes)
        plsc.subcore_barrier()

        def _rows_at(q):
            return a_hbm.at[pl.ds(q * _CH, _CH)]

        if nrem:
            @pl.when(wid < nrem)
            def _():
                pltpu.sync_copy(sd_hbm.at[nb * _NW + wid], ib0)
                pltpu.sync_copy(_rows_at(nb * _NW + wid), rows0)
                pltpu.sync_copy(rows0, acc_sh.at[ib0.at[1]], add=True)

        pltpu.sync_copy(sd_hbm.at[base], ib0)
        pltpu.sync_copy(sd_hbm.at[base + 1], ib1)
        pltpu.async_copy(_rows_at(base), rows0, gsem0)

        def pair(g, _):
            i0 = base + 2 * g
            pltpu.make_async_copy(_rows_at(i0), rows0, gsem0).wait()
            pltpu.async_copy(_rows_at(i0 + 1), rows1, gsem1)
            pltpu.sync_copy(rows0, acc_sh.at[ib0.at[1]], add=True)

            @pl.when(g < npair - 1)
            def _():
                pltpu.sync_copy(sd_hbm.at[i0 + 2], ib0)

            pltpu.make_async_copy(_rows_at(i0 + 1), rows1, gsem1).wait()

            @pl.when(g < npair - 1)
            def _():
                pltpu.async_copy(_rows_at(i0 + 2), rows0, gsem0)

            pltpu.sync_copy(rows1, acc_sh.at[ib1.at[1]], add=True)

            @pl.when(g < npair - 1)
            def _():
                pltpu.sync_copy(sd_hbm.at[i0 + 3], ib1)
            return 0

        lax.fori_loop(0, npair, pair, 0)

        plsc.subcore_barrier()
        _write_out(acc_sh, out_hbm.at[cid], sid, n_nodes)

    return lin_seg


_BR = 2000


def _init_body(v_ref, w_ref, b_ref, out_ref):
    h = jnp.dot(v_ref[...], w_ref[...], preferred_element_type=jnp.float32)
    out_ref[...] = jnp.maximum(h + b_ref[...], 0.0)


def _tc_init(V, W, b):
    n, dv = V.shape
    dh = W.shape[1]
    grid = n // _BR
    return pl.pallas_call(
        _init_body,
        grid=(grid,),
        in_specs=[
            pl.BlockSpec((_BR, dv), lambda i: (i, 0)),
            pl.BlockSpec((dv, dh), lambda i: (0, 0)),
            pl.BlockSpec((1, dh), lambda i: (0, 0)),
        ],
        out_specs=pl.BlockSpec((_BR, dh), lambda i: (i, 0)),
        out_shape=jax.ShapeDtypeStruct((n, dh), jnp.float32),
    )(V, W, b.reshape(1, dh))


_BE = 8000


def _edge_body(e_ref, w_ref, b_ref, out_ref):
    a = jnp.dot(e_ref[...], w_ref[...], preferred_element_type=jnp.float32)
    out_ref[...] = a + b_ref[...]


def _tc_edge_msg(E, We, b):
    ne, de = E.shape
    dh = We.shape[1]
    grid = ne // _BE
    return pl.pallas_call(
        _edge_body,
        grid=(grid,),
        in_specs=[
            pl.BlockSpec((_BE, de), lambda i: (i, 0)),
            pl.BlockSpec((de, dh), lambda i: (0, 0)),
            pl.BlockSpec((1, dh), lambda i: (0, 0)),
        ],
        out_specs=pl.BlockSpec((_BE, dh), lambda i: (i, 0)),
        out_shape=jax.ShapeDtypeStruct((ne, dh), jnp.float32),
    )(E, We, b.reshape(1, dh))


def _round_body(h0_ref, p_ref, wp_ref, s_ref, out_ref):
    p = p_ref[0] + p_ref[1]
    m = jnp.dot(p, wp_ref[...], preferred_element_type=jnp.float32)
    m = m + s_ref[0] + s_ref[1]
    out_ref[...] = jnp.maximum(h0_ref[...] + m, 0.0)


def _tc_round(H0, P, Wp, S):
    n, dh = H0.shape
    grid = n // _BR
    return pl.pallas_call(
        _round_body,
        grid=(grid,),
        in_specs=[
            pl.BlockSpec((_BR, dh), lambda i: (i, 0)),
            pl.BlockSpec((_NC, _BR, dh), lambda i: (0, i, 0)),
            pl.BlockSpec((dh, dh), lambda i: (0, 0)),
            pl.BlockSpec((_NC, _BR, dh), lambda i: (0, i, 0)),
        ],
        out_specs=pl.BlockSpec((_BR, dh), lambda i: (i, 0)),
        out_shape=jax.ShapeDtypeStruct((n, dh), jnp.float32),
    )(H0, P, Wp, S)


def _final_body(v_ref, wv_ref, p_ref, wp_ref, b_ref, out_ref):
    p = p_ref[0] + p_ref[1]
    h = jnp.dot(v_ref[...], wv_ref[...], preferred_element_type=jnp.float32)
    h = h + jnp.dot(p, wp_ref[...], preferred_element_type=jnp.float32)
    out_ref[...] = jnp.maximum(h + b_ref[...], 0.0)


def _tc_final(V, Wv, P, Wp, b):
    n, dv = V.shape
    dh = Wv.shape[1]
    grid = n // _BR
    return pl.pallas_call(
        _final_body,
        grid=(grid,),
        in_specs=[
            pl.BlockSpec((_BR, dv), lambda i: (i, 0)),
            pl.BlockSpec((dv, dh), lambda i: (0, 0)),
            pl.BlockSpec((_NC, _BR, dh), lambda i: (0, i, 0)),
            pl.BlockSpec((dh, dh), lambda i: (0, 0)),
            pl.BlockSpec((1, dh), lambda i: (0, 0)),
        ],
        out_specs=pl.BlockSpec((_BR, dh), lambda i: (i, 0)),
        out_shape=jax.ShapeDtypeStruct((n, dh), jnp.float32),
    )(V, Wv, P, Wp, b.reshape(1, dh))


def kernel(V, E, edge_index, rev_edge_index, batch, W_i, b_i, W_h, b_h,
           W_o, b_o):
    n, dv = V.shape
    ne, de = E.shape
    dh = W_i.shape[1]
    src = edge_index[0]
    dst = edge_index[1]

    zeros_h = jnp.zeros((_CH, dh), jnp.float32)

    ncht = ne // _CH
    sd = jnp.stack([src.reshape(ncht, _CH), dst.reshape(ncht, _CH)],
                   axis=1)

    lin_seg = _make_lin_seg(n, dh, ne)
    seg_sum = _make_seg_sum(n, dh, ne)

    Wp = W_h[:dh]
    H0 = _tc_init(V, W_i, b_i)
    P = seg_sum(H0, sd, zeros_h)
    A = _tc_edge_msg(E, W_h[dh:], b_h)
    S = lin_seg(A, sd, zeros_h)

    H = H0
    for r in range(_DEPTH - 1):
        if r > 0:
            P = seg_sum(H, sd, zeros_h)
        H = _tc_round(H0, P, Wp, S)

    P = seg_sum(H, sd, zeros_h)
    return _tc_final(V, W_o[:dv], P, W_o[dv:], b_o)


# speedup vs baseline: 1.2438x; 1.0086x over previous
import functools

import jax
import jax.numpy as jnp
from jax import lax
from jax.experimental import pallas as pl
from jax.experimental.pallas import tpu as pltpu
from jax.experimental.pallas import tpu_sc as plsc

_NC = 2
_NS = 16
_NW = _NC * _NS
_CH = 128
_DEPTH = 3


def _mesh():
    return plsc.VectorSubcoreMesh(
        core_axis_name="c", subcore_axis_name="s",
        num_cores=_NC, num_subcores=_NS)


def _part(n_nodes):
    base = n_nodes // (8 * _NS) * 8
    tail = n_nodes - base * _NS
    return base, tail


def _zero_acc(zeros_v, acc_sh, sid, n_nodes):
    rpt, tail = _part(n_nodes)
    start = sid * rpt
    for off in range(0, rpt, _CH):
        sz = min(_CH, rpt - off)
        pltpu.sync_copy(zeros_v.at[pl.ds(0, sz)],
                        acc_sh.at[pl.ds(start + off, sz)])
    if tail:
        @pl.when(sid == _NS - 1)
        def _():
            pltpu.sync_copy(zeros_v.at[pl.ds(0, tail)],
                            acc_sh.at[pl.ds(rpt * _NS, tail)])


def _write_out(acc_sh, out_ref, sid, n_nodes):
    rpt, tail = _part(n_nodes)
    pltpu.sync_copy(acc_sh.at[pl.ds(sid * rpt, rpt)],
                    out_ref.at[pl.ds(sid * rpt, rpt)])
    if tail:
        @pl.when(sid == _NS - 1)
        def _():
            pltpu.sync_copy(acc_sh.at[pl.ds(rpt * _NS, tail)],
                            out_ref.at[pl.ds(rpt * _NS, tail)])


def _make_seg_sum(n_nodes, d, n_edges):
    ncht = n_edges // _CH
    nb = ncht // _NW
    nrem = ncht - nb * _NW
    npair = nb // 2
    assert nb % 2 == 0 and n_edges == ncht * _CH

    @functools.partial(
        pl.kernel,
        out_type=jax.ShapeDtypeStruct((_NC, n_nodes, d), jnp.float32),
        mesh=_mesh(),
        scratch_types=[
            pltpu.VMEM((2, _CH), jnp.int32),
            pltpu.VMEM((2, _CH), jnp.int32),
            pltpu.VMEM((_CH, d), jnp.float32),
            pltpu.VMEM((_CH, d), jnp.float32),
            pltpu.VMEM_SHARED((n_nodes, d), jnp.float32),
            pltpu.SemaphoreType.DMA,
            pltpu.SemaphoreType.DMA,
        ],
    )
    def seg_sum(h_hbm, sd_hbm, zeros_hbm, out_hbm,
                ib0, ib1, rows0, rows1, acc_sh, gsem0, gsem1):
        cid = lax.axis_index("c")
        sid = lax.axis_index("s")
        wid = cid * _NS + sid
        base = wid * nb

        pltpu.sync_copy(zeros_hbm, rows0)
        _zero_acc(rows0, acc_sh, sid, n_nodes)
        plsc.subcore_barrier()

        if nrem:
            @pl.when(wid < nrem)
            def _():
                pltpu.sync_copy(sd_hbm.at[nb * _NW + wid], ib0)
                pltpu.async_copy(h_hbm.at[ib0.at[0]], rows0, gsem0).wait()
                pltpu.sync_copy(rows0, acc_sh.at[ib0.at[1]], add=True)

        pltpu.sync_copy(sd_hbm.at[base], ib0)
        pltpu.sync_copy(sd_hbm.at[base + 1], ib1)
        pltpu.async_copy(h_hbm.at[ib0.at[0]], rows0, gsem0)

        def pair(g, _):
            i0 = base + 2 * g
            pltpu.make_async_copy(h_hbm.at[ib0.at[0]], rows0, gsem0).wait()
            pltpu.async_copy(h_hbm.at[ib1.at[0]], rows1, gsem1)
            pltpu.sync_copy(rows0, acc_sh.at[ib0.at[1]], add=True)

            @pl.when(g < npair - 1)
            def _():
                pltpu.sync_copy(sd_hbm.at[i0 + 2], ib0)

            pltpu.make_async_copy(h_hbm.at[ib1.at[0]], rows1, gsem1).wait()

            @pl.when(g < npair - 1)
            def _():
                pltpu.async_copy(h_hbm.at[ib0.at[0]], rows0, gsem0)

            pltpu.sync_copy(rows1, acc_sh.at[ib1.at[1]], add=True)

            @pl.when(g < npair - 1)
            def _():
                pltpu.sync_copy(sd_hbm.at[i0 + 3], ib1)
            return 0

        lax.fori_loop(0, npair, pair, 0)

        plsc.subcore_barrier()
        _write_out(acc_sh, out_hbm.at[cid], sid, n_nodes)

    return seg_sum


def _make_lin_seg(n_nodes, d, n_edges):
    ncht = n_edges // _CH
    nb = ncht // _NW
    nrem = ncht - nb * _NW
    npair = nb // 2
    assert nb % 2 == 0 and n_edges == ncht * _CH

    @functools.partial(
        pl.kernel,
        out_type=jax.ShapeDtypeStruct((_NC, n_nodes, d), jnp.float32),
        mesh=_mesh(),
        scratch_types=[
            pltpu.VMEM((2, _CH), jnp.int32),
            pltpu.VMEM((2, _CH), jnp.int32),
            pltpu.VMEM((_CH, d), jnp.float32),
            pltpu.VMEM((_CH, d), jnp.float32),
            pltpu.VMEM_SHARED((n_nodes, d), jnp.float32),
            pltpu.SemaphoreType.DMA,
            pltpu.SemaphoreType.DMA,
        ],
    )
    def lin_seg(a_hbm, sd_hbm, zeros_hbm, out_hbm,
                ib0, ib1, rows0, rows1, acc_sh, gsem0, gsem1):
        cid = lax.axis_index("c")
        sid = lax.axis_index("s")
        wid = cid * _NS + sid
        base = wid * nb

        pltpu.sync_copy(zeros_hbm, rows0)
        _zero_acc(rows0, acc_sh, sid, n_nodes)
        plsc.subcore_barrier()

        def _rows_at(q):
            return a_hbm.at[pl.ds(q * _CH, _CH)]

        if nrem:
            @pl.when(wid < nrem)
            def _():
                pltpu.sync_copy(sd_hbm.at[nb * _NW + wid], ib0)
                pltpu.sync_copy(_rows_at(nb * _NW + wid), rows0)
                pltpu.sync_copy(rows0, acc_sh.at[ib0.at[1]], add=True)

        pltpu.sync_copy(sd_hbm.at[base], ib0)
        pltpu.sync_copy(sd_hbm.at[base + 1], ib1)
        pltpu.async_copy(_rows_at(base), rows0, gsem0)

        def pair(g, _):
            i0 = base + 2 * g
            pltpu.make_async_copy(_rows_at(i0), rows0, gsem0).wait()
            pltpu.async_copy(_rows_at(i0 + 1), rows1, gsem1)
            pltpu.sync_copy(rows0, acc_sh.at[ib0.at[1]], add=True)

            @pl.when(g < npair - 1)
            def _():
                pltpu.sync_copy(sd_hbm.at[i0 + 2], ib0)

            pltpu.make_async_copy(_rows_at(i0 + 1), rows1, gsem1).wait()

            @pl.when(g < npair - 1)
            def _():
                pltpu.async_copy(_rows_at(i0 + 2), rows0, gsem0)

            pltpu.sync_copy(rows1, acc_sh.at[ib1.at[1]], add=True)

            @pl.when(g < npair - 1)
            def _():
                pltpu.sync_copy(sd_hbm.at[i0 + 3], ib1)
            return 0

        lax.fori_loop(0, npair, pair, 0)

        plsc.subcore_barrier()
        _write_out(acc_sh, out_hbm.at[cid], sid, n_nodes)

    return lin_seg


_BR = 5000


def _init_body(v_ref, w_ref, b_ref, out_ref):
    h = jnp.dot(v_ref[...], w_ref[...], preferred_element_type=jnp.float32)
    out_ref[...] = jnp.maximum(h + b_ref[...], 0.0)


def _tc_init(V, W, b):
    n, dv = V.shape
    dh = W.shape[1]
    grid = n // _BR
    return pl.pallas_call(
        _init_body,
        grid=(grid,),
        in_specs=[
            pl.BlockSpec((_BR, dv), lambda i: (i, 0)),
            pl.BlockSpec((dv, dh), lambda i: (0, 0)),
            pl.BlockSpec((1, dh), lambda i: (0, 0)),
        ],
        out_specs=pl.BlockSpec((_BR, dh), lambda i: (i, 0)),
        out_shape=jax.ShapeDtypeStruct((n, dh), jnp.float32),
    )(V, W, b.reshape(1, dh))


_BE = 16000


def _edge_body(e_ref, w_ref, b_ref, out_ref):
    a = jnp.dot(e_ref[...], w_ref[...], preferred_element_type=jnp.float32)
    out_ref[...] = a + b_ref[...]


def _tc_edge_msg(E, We, b):
    ne, de = E.shape
    dh = We.shape[1]
    grid = ne // _BE
    return pl.pallas_call(
        _edge_body,
        grid=(grid,),
        in_specs=[
            pl.BlockSpec((_BE, de), lambda i: (i, 0)),
            pl.BlockSpec((de, dh), lambda i: (0, 0)),
            pl.BlockSpec((1, dh), lambda i: (0, 0)),
        ],
        out_specs=pl.BlockSpec((_BE, dh), lambda i: (i, 0)),
        out_shape=jax.ShapeDtypeStruct((ne, dh), jnp.float32),
    )(E, We, b.reshape(1, dh))


def _round_body(h0_ref, p_ref, wp_ref, s_ref, out_ref):
    p = p_ref[0] + p_ref[1]
    m = jnp.dot(p, wp_ref[...], preferred_element_type=jnp.float32)
    m = m + s_ref[0] + s_ref[1]
    out_ref[...] = jnp.maximum(h0_ref[...] + m, 0.0)


def _tc_round(H0, P, Wp, S):
    n, dh = H0.shape
    grid = n // _BR
    return pl.pallas_call(
        _round_body,
        grid=(grid,),
        in_specs=[
            pl.BlockSpec((_BR, dh), lambda i: (i, 0)),
            pl.BlockSpec((_NC, _BR, dh), lambda i: (0, i, 0)),
            pl.BlockSpec((dh, dh), lambda i: (0, 0)),
            pl.BlockSpec((_NC, _BR, dh), lambda i: (0, i, 0)),
        ],
        out_specs=pl.BlockSpec((_BR, dh), lambda i: (i, 0)),
        out_shape=jax.ShapeDtypeStruct((n, dh), jnp.float32),
    )(H0, P, Wp, S)


def _final_body(v_ref, wv_ref, p_ref, wp_ref, b_ref, out_ref):
    p = p_ref[0] + p_ref[1]
    h = jnp.dot(v_ref[...], wv_ref[...], preferred_element_type=jnp.float32)
    h = h + jnp.dot(p, wp_ref[...], preferred_element_type=jnp.float32)
    out_ref[...] = jnp.maximum(h + b_ref[...], 0.0)


def _tc_final(V, Wv, P, Wp, b):
    n, dv = V.shape
    dh = Wv.shape[1]
    grid = n // _BR
    return pl.pallas_call(
        _final_body,
        grid=(grid,),
        in_specs=[
            pl.BlockSpec((_BR, dv), lambda i: (i, 0)),
            pl.BlockSpec((dv, dh), lambda i: (0, 0)),
            pl.BlockSpec((_NC, _BR, dh), lambda i: (0, i, 0)),
            pl.BlockSpec((dh, dh), lambda i: (0, 0)),
            pl.BlockSpec((1, dh), lambda i: (0, 0)),
        ],
        out_specs=pl.BlockSpec((_BR, dh), lambda i: (i, 0)),
        out_shape=jax.ShapeDtypeStruct((n, dh), jnp.float32),
    )(V, Wv, P, Wp, b.reshape(1, dh))


def kernel(V, E, edge_index, rev_edge_index, batch, W_i, b_i, W_h, b_h,
           W_o, b_o):
    n, dv = V.shape
    ne, de = E.shape
    dh = W_i.shape[1]
    src = edge_index[0]
    dst = edge_index[1]

    zeros_h = jnp.zeros((_CH, dh), jnp.float32)

    ncht = ne // _CH
    sd = jnp.stack([src.reshape(ncht, _CH), dst.reshape(ncht, _CH)],
                   axis=1)

    lin_seg = _make_lin_seg(n, dh, ne)
    seg_sum = _make_seg_sum(n, dh, ne)

    Wp = W_h[:dh]
    H0 = _tc_init(V, W_i, b_i)
    P = seg_sum(H0, sd, zeros_h)
    A = _tc_edge_msg(E, W_h[dh:], b_h)
    S = lin_seg(A, sd, zeros_h)

    H = H0
    for r in range(_DEPTH - 1):
        if r > 0:
            P = seg_sum(H, sd, zeros_h)
        H = _tc_round(H0, P, Wp, S)

    P = seg_sum(H, sd, zeros_h)
    return _tc_final(V, W_o[:dv], P, W_o[dv:], b_o)
